# baseline (speedup 1.0000x reference)
import functools
import jax
import jax.numpy as jnp
from jax import lax
from jax.experimental import pallas as pl
from jax.experimental.pallas import tpu as pltpu
from jax.experimental.pallas import tpu_sc as plsc

N_NODES = 10000
N_EDGES = 320000
D = 128

NC = 2
NS = 16
NW = NC * NS
L = 16

N_PAD = 10240
CHUNK = 128
CH = 80
E_PAD = NW * CH * CHUNK
RPT = N_PAD // NS
W = 40
NWIN = CH // W

_mesh = plsc.VectorSubcoreMesh(core_axis_name="c", subcore_axis_name="s")


@functools.partial(
    pl.kernel,
    out_type=jax.ShapeDtypeStruct((NW * N_PAD,), jnp.float32),
    mesh=_mesh,
    scratch_types=[
        pltpu.VMEM((CH, CHUNK), jnp.int32),
        pltpu.VMEM((N_PAD,), jnp.float32),
    ],
    compiler_params=pltpu.CompilerParams(needs_layout_passes=False),
)
def _count_kernel(dst_hbm, out_hbm, dst_v, cnt_v):
    w = lax.axis_index("s") * NC + lax.axis_index("c")
    pltpu.sync_copy(dst_hbm.at[w], dst_v)

    zero16 = jnp.zeros((L,), jnp.float32)

    def zbody(i, _):
        cnt_v[pl.ds(i * L, L)] = zero16
        return 0

    lax.fori_loop(0, N_PAD // L, zbody, 0)

    one16 = jnp.ones((L,), jnp.float32)

    def row(j, _):
        def sub(k, _):
            d = dst_v[j, pl.ds(k * L, L)]
            plsc.addupdate_scatter(cnt_v, [d], one16)
            return 0
        lax.fori_loop(0, CHUNK // L, sub, 0)
        return 0

    lax.fori_loop(0, CH, row, 0)
    pltpu.sync_copy(cnt_v, out_hbm.at[pl.ds(w * N_PAD, N_PAD)])


NBUF = 2


@functools.partial(
    pl.kernel,
    out_type=jax.ShapeDtypeStruct((NC, N_PAD, D), jnp.float32),
    mesh=_mesh,
    scratch_types=[
        pltpu.VMEM((W, CHUNK), jnp.int32),
        pltpu.VMEM((W, CHUNK), jnp.int32),
        pltpu.VMEM((NBUF, CHUNK, D), jnp.float32),
        pltpu.VMEM_SHARED((N_PAD, D), jnp.float32),
        pltpu.SemaphoreType.DMA,
        pltpu.SemaphoreType.DMA,
    ],
)
def _agg_kernel(h_hbm, src_hbm, dst_hbm, zero_hbm, out_hbm,
                src_w, dst_w, rows_v, acc, sg0, sg1):
    sg = (sg0, sg1)
    c = lax.axis_index("c")
    s = lax.axis_index("s")
    w = s * NC + c
    @pl.when(c == 0)
    def _():
        pltpu.sync_copy(h_hbm.at[pl.ds(s * RPT, RPT)], acc.at[pl.ds(s * RPT, RPT)])

    @pl.when(c != 0)
    def _():
        pltpu.sync_copy(zero_hbm.at[pl.ds(s * RPT, RPT)], acc.at[pl.ds(s * RPT, RPT)])

    plsc.subcore_barrier()

    for win in range(NWIN):
        pltpu.sync_copy(src_hbm.at[w, pl.ds(win * W, W)], src_w)
        pltpu.sync_copy(dst_hbm.at[w, pl.ds(win * W, W)], dst_w)

        def inner(t, _):
            j0 = t * NBUF
            d0 = pltpu.async_copy(h_hbm.at[src_w.at[j0]], rows_v.at[0], sg0)
            d1 = pltpu.async_copy(h_hbm.at[src_w.at[j0 + 1]], rows_v.at[1], sg1)
            d0.wait()
            pltpu.sync_copy(rows_v.at[0], acc.at[dst_w.at[j0]], add=True)
            d1.wait()
            pltpu.sync_copy(rows_v.at[1], acc.at[dst_w.at[j0 + 1]], add=True)
            return 0

        lax.fori_loop(0, W // NBUF, inner, 0)
    plsc.subcore_barrier()
    pltpu.sync_copy(acc.at[pl.ds(s * RPT, RPT)], out_hbm.at[c, pl.ds(s * RPT, RPT)])


BLK = 1024


def _dinv_of(cnt_blk):
    deg = jnp.sum(cnt_blk, axis=0) + 1.0
    return lax.rsqrt(deg)


def _mm1_body(cnt_ref, x_ref, w_ref, h_ref):
    dinv = _dinv_of(cnt_ref[...])
    h = jnp.dot(x_ref[...], w_ref[...], preferred_element_type=jnp.float32)
    h_ref[...] = h * dinv[:, None]


def _mid_body(cnt_ref, p_ref, b1_ref, w_ref, x1_ref, h2_ref):
    dinv = _dinv_of(cnt_ref[...])
    agg = p_ref[0] + p_ref[1]
    x1 = jnp.maximum(agg * dinv[:, None] + b1_ref[...], 0.0)
    x1_ref[...] = x1
    h2 = jnp.dot(x1, w_ref[...], preferred_element_type=jnp.float32)
    h2_ref[...] = h2 * dinv[:, None]


def _fin_body(cnt_ref, p_ref, b2_ref, x2_ref):
    dinv = _dinv_of(cnt_ref[...])
    agg = p_ref[0] + p_ref[1]
    x2_ref[...] = agg * dinv[:, None] + b2_ref[...]


_cnt_spec = pl.BlockSpec((NW, BLK), lambda i: (0, i))
_row_spec = pl.BlockSpec((BLK, D), lambda i: (i, 0))
_par_spec = pl.BlockSpec((NC, BLK, D), lambda i: (0, i, 0))
_w_spec = pl.BlockSpec((D, D), lambda i: (0, 0))
_b_spec = pl.BlockSpec((1, D), lambda i: (0, 0))
_grid = (N_PAD // BLK,)

_mm1 = pl.pallas_call(
    _mm1_body,
    grid=_grid,
    in_specs=[_cnt_spec, _row_spec, _w_spec],
    out_specs=_row_spec,
    out_shape=jax.ShapeDtypeStruct((N_PAD, D), jnp.float32),
)

_mid = pl.pallas_call(
    _mid_body,
    grid=_grid,
    in_specs=[_cnt_spec, _par_spec, _b_spec, _w_spec],
    out_specs=[_row_spec, _row_spec],
    out_shape=[
        jax.ShapeDtypeStruct((N_PAD, D), jnp.float32),
        jax.ShapeDtypeStruct((N_PAD, D), jnp.float32),
    ],
)

_fin = pl.pallas_call(
    _fin_body,
    grid=_grid,
    in_specs=[_cnt_spec, _par_spec, _b_spec],
    out_specs=_row_spec,
    out_shape=jax.ShapeDtypeStruct((N_PAD, D), jnp.float32),
)


@jax.jit
def kernel(x, edge_index, W1, b1, W2, b2):
    n_fill = E_PAD - N_EDGES
    fill_dst = N_NODES + (jnp.arange(n_fill, dtype=jnp.int32) % (N_PAD - N_NODES))
    src = jnp.concatenate(
        [edge_index[0], jnp.zeros((n_fill,), jnp.int32)]
    ).reshape(NW, CH, CHUNK)
    dst = jnp.concatenate([edge_index[1], fill_dst]).reshape(NW, CH, CHUNK)
    x_pad = jnp.zeros((N_PAD, D), jnp.float32).at[:N_NODES].set(x)
    zeros_init = jnp.zeros((N_PAD, D), jnp.float32)

    cnt_parts = _count_kernel(dst).reshape(NW, N_PAD)
    h1 = _mm1(cnt_parts, x_pad, W1)
    p1 = _agg_kernel(h1, src, dst, zeros_init)
    x1_pad, h2 = _mid(cnt_parts, p1, b1.reshape(1, D), W2)
    p2 = _agg_kernel(h2, src, dst, zeros_init)
    x2_pad = _fin(cnt_parts, p2, b2.reshape(1, D))
    return (x1_pad[:N_NODES], x2_pad[:N_NODES])


# speedup vs baseline: 1.8407x; 1.8407x over previous
import functools
import jax
import jax.numpy as jnp
from jax import lax
from jax.experimental import pallas as pl
from jax.experimental.pallas import tpu as pltpu
from jax.experimental.pallas import tpu_sc as plsc

N_NODES = 10000
N_EDGES = 320000
D = 128

NC = 2
NS = 16
NW = NC * NS
L = 16

N_PAD = 10240
CHUNK = 128
E_TOT = N_EDGES + N_NODES
CH = -(-E_TOT // (NW * CHUNK))
E_PAD = NW * CH * CHUNK
RPT = N_PAD // NS

_mesh = plsc.VectorSubcoreMesh(core_axis_name="c", subcore_axis_name="s")


@functools.partial(
    pl.kernel,
    out_type=jax.ShapeDtypeStruct((NW * N_PAD,), jnp.float32),
    mesh=_mesh,
    scratch_types=[
        pltpu.VMEM((CH, CHUNK), jnp.int32),
        pltpu.VMEM((N_PAD,), jnp.float32),
    ],
    compiler_params=pltpu.CompilerParams(needs_layout_passes=False),
)
def _count_kernel(dst_hbm, out_hbm, dst_v, cnt_v):
    w = lax.axis_index("s") * NC + lax.axis_index("c")
    pltpu.sync_copy(dst_hbm.at[w], dst_v)

    zero16 = jnp.zeros((L,), jnp.float32)

    def zbody(i, _):
        cnt_v[pl.ds(i * L, L)] = zero16
        return 0

    lax.fori_loop(0, N_PAD // L, zbody, 0)

    one16 = jnp.ones((L,), jnp.float32)

    def row(j, _):
        def sub(k, _):
            d = dst_v[j, pl.ds(k * L, L)]
            plsc.addupdate_scatter(cnt_v, [d], one16)
            return 0
        lax.fori_loop(0, CHUNK // L, sub, 0)
        return 0

    lax.fori_loop(0, CH, row, 0)
    pltpu.sync_copy(cnt_v, out_hbm.at[pl.ds(w * N_PAD, N_PAD)])


@functools.partial(
    pl.kernel,
    out_type=jax.ShapeDtypeStruct((NC, N_PAD, D), jnp.float32),
    mesh=_mesh,
    scratch_types=[
        pltpu.VMEM((CH, CHUNK), jnp.int32),
        pltpu.VMEM((CH, CHUNK), jnp.int32),
        pltpu.VMEM((CHUNK, D), jnp.float32),
        pltpu.VMEM_SHARED((N_PAD, D), jnp.float32),
        pltpu.SemaphoreType.DMA,
    ],
)
def _agg_kernel(h_hbm, src_hbm, dst_hbm, zero_hbm, out_hbm,
                src_v, dst_v, rows_v, acc, sem):
    c = lax.axis_index("c")
    s = lax.axis_index("s")
    w = s * NC + c
    pltpu.sync_copy(src_hbm.at[w], src_v)
    pltpu.sync_copy(dst_hbm.at[w], dst_v)
    pltpu.sync_copy(zero_hbm.at[pl.ds(s * RPT, RPT)], acc.at[pl.ds(s * RPT, RPT)])
    plsc.subcore_barrier()

    def chunk(j, _):
        pltpu.async_copy(h_hbm.at[src_v.at[j]], rows_v, sem).wait()
        pltpu.sync_copy(rows_v, acc.at[dst_v.at[j]], add=True)
        return 0

    lax.fori_loop(0, CH, chunk, 0)
    plsc.subcore_barrier()
    pltpu.sync_copy(acc.at[pl.ds(s * RPT, RPT)], out_hbm.at[c, pl.ds(s * RPT, RPT)])


BLK = 1024


def _dinv_of(cnt_blk):
    deg = jnp.sum(cnt_blk, axis=0)
    return lax.rsqrt(jnp.maximum(deg, 1.0))


def _mm1_body(cnt_ref, x_ref, w_ref, h_ref):
    dinv = _dinv_of(cnt_ref[...])
    h = jnp.dot(x_ref[...], w_ref[...], preferred_element_type=jnp.float32)
    h_ref[...] = h * dinv[:, None]


def _mid_body(cnt_ref, p_ref, b1_ref, w_ref, x1_ref, h2_ref):
    dinv = _dinv_of(cnt_ref[...])
    agg = p_ref[0] + p_ref[1]
    x1 = jnp.maximum(agg * dinv[:, None] + b1_ref[...], 0.0)
    x1_ref[...] = x1
    h2 = jnp.dot(x1, w_ref[...], preferred_element_type=jnp.float32)
    h2_ref[...] = h2 * dinv[:, None]


def _fin_body(cnt_ref, p_ref, b2_ref, x2_ref):
    dinv = _dinv_of(cnt_ref[...])
    agg = p_ref[0] + p_ref[1]
    x2_ref[...] = agg * dinv[:, None] + b2_ref[...]


_cnt_spec = pl.BlockSpec((NW, BLK), lambda i: (0, i))
_row_spec = pl.BlockSpec((BLK, D), lambda i: (i, 0))
_par_spec = pl.BlockSpec((NC, BLK, D), lambda i: (0, i, 0))
_w_spec = pl.BlockSpec((D, D), lambda i: (0, 0))
_b_spec = pl.BlockSpec((1, D), lambda i: (0, 0))
_grid = (N_PAD // BLK,)

_mm1 = pl.pallas_call(
    _mm1_body,
    grid=_grid,
    in_specs=[_cnt_spec, _row_spec, _w_spec],
    out_specs=_row_spec,
    out_shape=jax.ShapeDtypeStruct((N_PAD, D), jnp.float32),
)

_mid = pl.pallas_call(
    _mid_body,
    grid=_grid,
    in_specs=[_cnt_spec, _par_spec, _b_spec, _w_spec],
    out_specs=[_row_spec, _row_spec],
    out_shape=[
        jax.ShapeDtypeStruct((N_PAD, D), jnp.float32),
        jax.ShapeDtypeStruct((N_PAD, D), jnp.float32),
    ],
)

_fin = pl.pallas_call(
    _fin_body,
    grid=_grid,
    in_specs=[_cnt_spec, _par_spec, _b_spec],
    out_specs=_row_spec,
    out_shape=jax.ShapeDtypeStruct((N_PAD, D), jnp.float32),
)


@jax.jit
def kernel(x, edge_index, W1, b1, W2, b2):
    loop = jnp.arange(N_NODES, dtype=jnp.int32)
    n_fill = E_PAD - E_TOT
    src = jnp.concatenate(
        [edge_index[0], loop, jnp.zeros((n_fill,), jnp.int32)]
    ).reshape(NW, CH, CHUNK)
    dst = jnp.concatenate(
        [edge_index[1], loop, jnp.full((n_fill,), N_NODES, jnp.int32)]
    ).reshape(NW, CH, CHUNK)
    x_pad = jnp.zeros((N_PAD, D), jnp.float32).at[:N_NODES].set(x)
    zeros_init = jnp.zeros((N_PAD, D), jnp.float32)

    cnt_parts = _count_kernel(dst).reshape(NW, N_PAD)
    h1 = _mm1(cnt_parts, x_pad, W1)
    p1 = _agg_kernel(h1, src, dst, zeros_init)
    x1_pad, h2 = _mid(cnt_parts, p1, b1.reshape(1, D), W2)
    p2 = _agg_kernel(h2, src, dst, zeros_init)
    x2_pad = _fin(cnt_parts, p2, b2.reshape(1, D))
    return (x1_pad[:N_NODES], x2_pad[:N_NODES])


# speedup vs baseline: 2.3451x; 1.2740x over previous
import functools
import jax
import jax.numpy as jnp
from jax import lax
from jax.experimental import pallas as pl
from jax.experimental.pallas import tpu as pltpu
from jax.experimental.pallas import tpu_sc as plsc

N_NODES = 10000
N_EDGES = 320000
D = 128

NC = 2
NS = 16
NW = NC * NS
L = 16

N_PAD = 10240
CHUNK = 128
E_TOT = N_EDGES + N_NODES
CH = -(-E_TOT // (NW * CHUNK))
E_PAD = NW * CH * CHUNK
RPT = N_PAD // NS

_mesh = plsc.VectorSubcoreMesh(core_axis_name="c", subcore_axis_name="s")


@functools.partial(
    pl.kernel,
    out_type=jax.ShapeDtypeStruct((NW * N_PAD,), jnp.float32),
    mesh=_mesh,
    scratch_types=[
        pltpu.VMEM((CH, CHUNK), jnp.int32),
        pltpu.VMEM((N_PAD,), jnp.float32),
    ],
    compiler_params=pltpu.CompilerParams(needs_layout_passes=False),
)
def _count_kernel(dst_hbm, out_hbm, dst_v, cnt_v):
    w = lax.axis_index("s") * NC + lax.axis_index("c")
    pltpu.sync_copy(dst_hbm.at[w], dst_v)

    zero16 = jnp.zeros((L,), jnp.float32)

    def zbody(i, _):
        cnt_v[pl.ds(i * L, L)] = zero16
        return 0

    lax.fori_loop(0, N_PAD // L, zbody, 0)

    one16 = jnp.ones((L,), jnp.float32)

    def row(j, _):
        def sub(k, _):
            d = dst_v[j, pl.ds(k * L, L)]
            plsc.addupdate_scatter(cnt_v, [d], one16)
            return 0
        lax.fori_loop(0, CHUNK // L, sub, 0)
        return 0

    lax.fori_loop(0, CH, row, 0)
    pltpu.sync_copy(cnt_v, out_hbm.at[pl.ds(w * N_PAD, N_PAD)])


@functools.partial(
    pl.kernel,
    out_type=jax.ShapeDtypeStruct((NC, N_PAD, D), jnp.float32),
    mesh=_mesh,
    scratch_types=[
        pltpu.VMEM((CH, CHUNK), jnp.int32),
        pltpu.VMEM((CH, CHUNK), jnp.int32),
        pltpu.VMEM((CHUNK, D), jnp.float32),
        pltpu.VMEM_SHARED((N_PAD, D), jnp.float32),
        pltpu.SemaphoreType.DMA,
    ],
)
def _agg_kernel(h_hbm, src_hbm, dst_hbm, zero_hbm, out_hbm,
                src_v, dst_v, rows_v, acc, sem):
    c = lax.axis_index("c")
    s = lax.axis_index("s")
    w = s * NC + c
    pltpu.sync_copy(src_hbm.at[w], src_v)
    pltpu.sync_copy(dst_hbm.at[w], dst_v)
    pltpu.sync_copy(zero_hbm.at[pl.ds(s * RPT, RPT)], acc.at[pl.ds(s * RPT, RPT)])
    plsc.subcore_barrier()

    def chunk(j, _):
        pltpu.async_copy(h_hbm.at[src_v.at[j]], rows_v, sem).wait()
        pltpu.sync_copy(rows_v, acc.at[dst_v.at[j]], add=True)
        return 0

    lax.fori_loop(0, CH, chunk, 0)
    plsc.subcore_barrier()
    pltpu.sync_copy(acc.at[pl.ds(s * RPT, RPT)], out_hbm.at[c, pl.ds(s * RPT, RPT)])


BLK = 1024


def _dinv_of(cnt_blk):
    deg = jnp.sum(cnt_blk, axis=0)
    return lax.rsqrt(jnp.maximum(deg, 1.0))


def _mm1_body(cnt_ref, x_ref, w_ref, h_ref):
    dinv = _dinv_of(cnt_ref[...])
    h = jnp.dot(x_ref[...], w_ref[...], preferred_element_type=jnp.float32)
    h_ref[...] = h * dinv[:, None]


def _mid_body(cnt_ref, p_ref, b1_ref, w_ref, x1_ref, h2_ref):
    dinv = _dinv_of(cnt_ref[...])
    agg = p_ref[0] + p_ref[1]
    x1 = jnp.maximum(agg * dinv[:, None] + b1_ref[...], 0.0)
    x1_ref[...] = x1
    h2 = jnp.dot(x1, w_ref[...], preferred_element_type=jnp.float32)
    h2_ref[...] = h2 * dinv[:, None]


def _fin_body(cnt_ref, p_ref, b2_ref, x2_ref):
    dinv = _dinv_of(cnt_ref[...])
    agg = p_ref[0] + p_ref[1]
    x2_ref[...] = agg * dinv[:, None] + b2_ref[...]


_cnt_spec = pl.BlockSpec((NW, BLK), lambda i: (0, i))
_row_spec = pl.BlockSpec((BLK, D), lambda i: (i, 0))
_par_spec = pl.BlockSpec((NC, BLK, D), lambda i: (0, i, 0))
_w_spec = pl.BlockSpec((D, D), lambda i: (0, 0))
_b_spec = pl.BlockSpec((1, D), lambda i: (0, 0))
_grid = (N_PAD // BLK,)

_mm1 = pl.pallas_call(
    _mm1_body,
    grid=_grid,
    in_specs=[_cnt_spec, _row_spec, _w_spec],
    out_specs=_row_spec,
    out_shape=jax.ShapeDtypeStruct((N_PAD, D), jnp.float32),
)

_mid = pl.pallas_call(
    _mid_body,
    grid=_grid,
    in_specs=[_cnt_spec, _par_spec, _b_spec, _w_spec],
    out_specs=[_row_spec, _row_spec],
    out_shape=[
        jax.ShapeDtypeStruct((N_PAD, D), jnp.float32),
        jax.ShapeDtypeStruct((N_PAD, D), jnp.float32),
    ],
)

_fin = pl.pallas_call(
    _fin_body,
    grid=_grid,
    in_specs=[_cnt_spec, _par_spec, _b_spec],
    out_specs=_row_spec,
    out_shape=jax.ShapeDtypeStruct((N_PAD, D), jnp.float32),
)


@jax.jit
def kernel(x, edge_index, W1, b1, W2, b2):
    loop = jnp.arange(N_NODES, dtype=jnp.int32)
    n_fill = E_PAD - E_TOT
    fill = jnp.arange(n_fill, dtype=jnp.int32)
    src = jnp.concatenate(
        [edge_index[0], loop, fill % N_NODES]
    ).reshape(NW, CH, CHUNK)
    dst = jnp.concatenate(
        [edge_index[1], loop, N_NODES + (fill % (N_PAD - N_NODES))]
    ).reshape(NW, CH, CHUNK)
    x_pad = jnp.zeros((N_PAD, D), jnp.float32).at[:N_NODES].set(x)
    zeros_init = jnp.zeros((N_PAD, D), jnp.float32)

    cnt_parts = _count_kernel(dst).reshape(NW, N_PAD)
    h1 = _mm1(cnt_parts, x_pad, W1)
    p1 = _agg_kernel(h1, src, dst, zeros_init)
    x1_pad, h2 = _mid(cnt_parts, p1, b1.reshape(1, D), W2)
    p2 = _agg_kernel(h2, src, dst, zeros_init)
    x2_pad = _fin(cnt_parts, p2, b2.reshape(1, D))
    return (x1_pad[:N_NODES], x2_pad[:N_NODES])


# speedup vs baseline: 2.7012x; 1.1519x over previous
import functools
import jax
import jax.numpy as jnp
from jax import lax
from jax.experimental import pallas as pl
from jax.experimental.pallas import tpu as pltpu
from jax.experimental.pallas import tpu_sc as plsc

N_NODES = 10000
N_EDGES = 320000
D = 128

NC = 2
NS = 16
NW = NC * NS
L = 16

N_PAD = 10240
CHUNK = 128
CH = 80
E_PAD = NW * CH * CHUNK
RPT = N_PAD // NS
W = 40
NWIN = CH // W
NBUF = 2

_mesh = plsc.VectorSubcoreMesh(core_axis_name="c", subcore_axis_name="s")


@functools.partial(
    pl.kernel,
    out_type=jax.ShapeDtypeStruct((NW * N_PAD,), jnp.float32),
    mesh=_mesh,
    scratch_types=[
        pltpu.VMEM((CH, CHUNK), jnp.int32),
        pltpu.VMEM((N_PAD,), jnp.float32),
    ],
    compiler_params=pltpu.CompilerParams(needs_layout_passes=False),
)
def _count_kernel(dst_hbm, out_hbm, dst_v, cnt_v):
    w = lax.axis_index("s") * NC + lax.axis_index("c")
    pltpu.sync_copy(dst_hbm.at[w], dst_v)

    zero16 = jnp.zeros((L,), jnp.float32)

    def zbody(i, _):
        cnt_v[pl.ds(i * L, L)] = zero16
        return 0

    lax.fori_loop(0, N_PAD // L, zbody, 0)

    one16 = jnp.ones((L,), jnp.float32)

    def row(j, _):
        def sub(k, _):
            d = dst_v[j, pl.ds(k * L, L)]
            plsc.addupdate_scatter(cnt_v, [d], one16)
            return 0
        lax.fori_loop(0, CHUNK // L, sub, 0)
        return 0

    lax.fori_loop(0, CH, row, 0)
    pltpu.sync_copy(cnt_v, out_hbm.at[pl.ds(w * N_PAD, N_PAD)])


@functools.partial(
    pl.kernel,
    out_type=jax.ShapeDtypeStruct((NC, N_PAD, D), jnp.float32),
    mesh=_mesh,
    scratch_types=[
        pltpu.VMEM((W, CHUNK), jnp.int32),
        pltpu.VMEM((W, CHUNK), jnp.int32),
        pltpu.VMEM((NBUF, CHUNK, D), jnp.float32),
        pltpu.VMEM_SHARED((N_PAD, D), jnp.float32),
        pltpu.SemaphoreType.DMA,
        pltpu.SemaphoreType.DMA,
    ],
)
def _agg_kernel(h_hbm, src_hbm, dst_hbm, zero_hbm, out_hbm,
                src_w, dst_w, rows_v, acc, sg0, sg1):
    c = lax.axis_index("c")
    s = lax.axis_index("s")
    w = s * NC + c
    @pl.when(c == 0)
    def _():
        pltpu.sync_copy(h_hbm.at[pl.ds(s * RPT, RPT)], acc.at[pl.ds(s * RPT, RPT)])

    @pl.when(c != 0)
    def _():
        pltpu.sync_copy(zero_hbm.at[pl.ds(s * RPT, RPT)], acc.at[pl.ds(s * RPT, RPT)])

    plsc.subcore_barrier()

    for win in range(NWIN):
        pltpu.sync_copy(src_hbm.at[w, pl.ds(win * W, W)], src_w)
        pltpu.sync_copy(dst_hbm.at[w, pl.ds(win * W, W)], dst_w)

        def inner(t, _):
            j0 = t * NBUF
            d0 = pltpu.async_copy(h_hbm.at[src_w.at[j0]], rows_v.at[0], sg0)
            d1 = pltpu.async_copy(h_hbm.at[src_w.at[j0 + 1]], rows_v.at[1], sg1)
            d0.wait()
            pltpu.sync_copy(rows_v.at[0], acc.at[dst_w.at[j0]], add=True)
            d1.wait()
            pltpu.sync_copy(rows_v.at[1], acc.at[dst_w.at[j0 + 1]], add=True)
            return 0

        lax.fori_loop(0, W // NBUF, inner, 0)
    plsc.subcore_barrier()
    pltpu.sync_copy(acc.at[pl.ds(s * RPT, RPT)], out_hbm.at[c, pl.ds(s * RPT, RPT)])


BLK = 1024


def _dinv_of(cnt_blk):
    deg = jnp.sum(cnt_blk, axis=0) + 1.0
    return lax.rsqrt(deg)


def _mm1_body(cnt_ref, x_ref, w_ref, h_ref):
    dinv = _dinv_of(cnt_ref[...])
    h = jnp.dot(x_ref[...], w_ref[...], preferred_element_type=jnp.float32)
    h_ref[...] = h * dinv[:, None]


def _mid_body(cnt_ref, p_ref, b1_ref, w_ref, x1_ref, h2_ref):
    dinv = _dinv_of(cnt_ref[...])
    agg = p_ref[0] + p_ref[1]
    x1 = jnp.maximum(agg * dinv[:, None] + b1_ref[...], 0.0)
    x1_ref[...] = x1
    h2 = jnp.dot(x1, w_ref[...], preferred_element_type=jnp.float32)
    h2_ref[...] = h2 * dinv[:, None]


def _fin_body(cnt_ref, p_ref, b2_ref, x2_ref):
    dinv = _dinv_of(cnt_ref[...])
    agg = p_ref[0] + p_ref[1]
    x2_ref[...] = agg * dinv[:, None] + b2_ref[...]


_cnt_spec = pl.BlockSpec((NW, BLK), lambda i: (0, i))
_row_spec = pl.BlockSpec((BLK, D), lambda i: (i, 0))
_par_spec = pl.BlockSpec((NC, BLK, D), lambda i: (0, i, 0))
_w_spec = pl.BlockSpec((D, D), lambda i: (0, 0))
_b_spec = pl.BlockSpec((1, D), lambda i: (0, 0))
_grid = (N_PAD // BLK,)

_mm1 = pl.pallas_call(
    _mm1_body,
    grid=_grid,
    in_specs=[_cnt_spec, _row_spec, _w_spec],
    out_specs=_row_spec,
    out_shape=jax.ShapeDtypeStruct((N_PAD, D), jnp.float32),
)

_mid = pl.pallas_call(
    _mid_body,
    grid=_grid,
    in_specs=[_cnt_spec, _par_spec, _b_spec, _w_spec],
    out_specs=[_row_spec, _row_spec],
    out_shape=[
        jax.ShapeDtypeStruct((N_PAD, D), jnp.float32),
        jax.ShapeDtypeStruct((N_PAD, D), jnp.float32),
    ],
)

_fin = pl.pallas_call(
    _fin_body,
    grid=_grid,
    in_specs=[_cnt_spec, _par_spec, _b_spec],
    out_specs=_row_spec,
    out_shape=jax.ShapeDtypeStruct((N_PAD, D), jnp.float32),
)


@jax.jit
def kernel(x, edge_index, W1, b1, W2, b2):
    n_fill = E_PAD - N_EDGES
    fill = jnp.arange(n_fill, dtype=jnp.int32)
    src = jnp.concatenate(
        [edge_index[0], fill % N_NODES]
    ).reshape(NW, CH, CHUNK)
    dst = jnp.concatenate(
        [edge_index[1], N_NODES + (fill % (N_PAD - N_NODES))]
    ).reshape(NW, CH, CHUNK)
    x_pad = jnp.zeros((N_PAD, D), jnp.float32).at[:N_NODES].set(x)
    zeros_init = jnp.zeros((N_PAD, D), jnp.float32)

    cnt_parts = _count_kernel(dst).reshape(NW, N_PAD)
    h1 = _mm1(cnt_parts, x_pad, W1)
    p1 = _agg_kernel(h1, src, dst, zeros_init)
    x1_pad, h2 = _mid(cnt_parts, p1, b1.reshape(1, D), W2)
    p2 = _agg_kernel(h2, src, dst, zeros_init)
    x2_pad = _fin(cnt_parts, p2, b2.reshape(1, D))
    return (x1_pad[:N_NODES], x2_pad[:N_NODES])


# speedup vs baseline: 3.4277x; 1.2690x over previous
import functools
import jax
import jax.numpy as jnp
from jax import lax
from jax.experimental import pallas as pl
from jax.experimental.pallas import tpu as pltpu
from jax.experimental.pallas import tpu_sc as plsc

N_NODES = 10000
N_EDGES = 320000
D = 128

NC = 2
NS = 16
NW = NC * NS
L = 16

N_PAD = 10240
CHUNK = 128
CH = 80
E_PAD = NW * CH * CHUNK
RPT = N_PAD // NS
W = 40
NWIN = CH // W
NBUF = 2

_mesh = plsc.VectorSubcoreMesh(core_axis_name="c", subcore_axis_name="s")


@functools.partial(
    pl.kernel,
    out_type=jax.ShapeDtypeStruct((NW * N_PAD,), jnp.float32),
    mesh=_mesh,
    scratch_types=[
        pltpu.VMEM((CH, CHUNK), jnp.int32),
        pltpu.VMEM((N_PAD,), jnp.float32),
    ],
    compiler_params=pltpu.CompilerParams(needs_layout_passes=False),
)
def _count_kernel(dst_hbm, out_hbm, dst_v, cnt_v):
    w = lax.axis_index("s") * NC + lax.axis_index("c")
    pltpu.sync_copy(dst_hbm.at[w], dst_v)

    zero16 = jnp.zeros((L,), jnp.float32)

    def zbody(i, _):
        cnt_v[pl.ds(i * L, L)] = zero16
        return 0

    lax.fori_loop(0, N_PAD // L, zbody, 0)

    one16 = jnp.ones((L,), jnp.float32)

    def row(j, _):
        def sub(k, _):
            d = dst_v[j, pl.ds(k * L, L)]
            plsc.addupdate_scatter(cnt_v, [d], one16)
            return 0
        lax.fori_loop(0, CHUNK // L, sub, 0)
        return 0

    lax.fori_loop(0, CH, row, 0)
    pltpu.sync_copy(cnt_v, out_hbm.at[pl.ds(w * N_PAD, N_PAD)])


@functools.partial(
    pl.kernel,
    out_type=jax.ShapeDtypeStruct((NC, N_PAD, D), jnp.float32),
    mesh=_mesh,
    scratch_types=[
        pltpu.VMEM((W, CHUNK), jnp.int32),
        pltpu.VMEM((W, CHUNK), jnp.int32),
        pltpu.VMEM((NBUF, CHUNK, D), jnp.float32),
        pltpu.VMEM_SHARED((N_PAD, D), jnp.float32),
        pltpu.SemaphoreType.DMA,
        pltpu.SemaphoreType.DMA,
    ],
)
def _agg_kernel(h_hbm, src_hbm, dst_hbm, zero_hbm, out_hbm,
                src_w, dst_w, rows_v, acc, sg0, sg1):
    c = lax.axis_index("c")
    s = lax.axis_index("s")
    w = s * NC + c
    @pl.when(c == 0)
    def _():
        pltpu.sync_copy(h_hbm.at[pl.ds(s * RPT, RPT)], acc.at[pl.ds(s * RPT, RPT)])

    @pl.when(c != 0)
    def _():
        pltpu.sync_copy(zero_hbm.at[pl.ds(s * RPT, RPT)], acc.at[pl.ds(s * RPT, RPT)])

    plsc.subcore_barrier()

    for win in range(NWIN):
        pltpu.sync_copy(src_hbm.at[w, pl.ds(win * W, W)], src_w)
        pltpu.sync_copy(dst_hbm.at[w, pl.ds(win * W, W)], dst_w)

        sg = (sg0, sg1)
        for b in range(NBUF):
            pltpu.async_copy(h_hbm.at[src_w.at[b]], rows_v.at[b], sg[b])

        def inner(t, _):
            j0 = t * NBUF
            for b in range(NBUF):
                j = j0 + b
                pltpu.make_async_copy(
                    h_hbm.at[src_w.at[j]], rows_v.at[b], sg[b]
                ).wait()
                pltpu.sync_copy(rows_v.at[b], acc.at[dst_w.at[j]], add=True)

                @pl.when(j + NBUF < W)
                def _():
                    pltpu.async_copy(
                        h_hbm.at[src_w.at[j + NBUF]], rows_v.at[b], sg[b]
                    )
            return 0

        lax.fori_loop(0, W // NBUF, inner, 0)
    plsc.subcore_barrier()
    pltpu.sync_copy(acc.at[pl.ds(s * RPT, RPT)], out_hbm.at[c, pl.ds(s * RPT, RPT)])


BLK = 1024


def _dinv_of(cnt_blk):
    deg = jnp.sum(cnt_blk, axis=0) + 1.0
    return lax.rsqrt(deg)


def _mm1_body(cnt_ref, x_ref, w_ref, h_ref):
    dinv = _dinv_of(cnt_ref[...])
    h = jnp.dot(x_ref[...], w_ref[...], preferred_element_type=jnp.float32)
    h_ref[...] = h * dinv[:, None]


def _mid_body(cnt_ref, p_ref, b1_ref, w_ref, x1_ref, h2_ref):
    dinv = _dinv_of(cnt_ref[...])
    agg = p_ref[0] + p_ref[1]
    x1 = jnp.maximum(agg * dinv[:, None] + b1_ref[...], 0.0)
    x1_ref[...] = x1
    h2 = jnp.dot(x1, w_ref[...], preferred_element_type=jnp.float32)
    h2_ref[...] = h2 * dinv[:, None]


def _fin_body(cnt_ref, p_ref, b2_ref, x2_ref):
    dinv = _dinv_of(cnt_ref[...])
    agg = p_ref[0] + p_ref[1]
    x2_ref[...] = agg * dinv[:, None] + b2_ref[...]


_cnt_spec = pl.BlockSpec((NW, BLK), lambda i: (0, i))
_row_spec = pl.BlockSpec((BLK, D), lambda i: (i, 0))
_par_spec = pl.BlockSpec((NC, BLK, D), lambda i: (0, i, 0))
_w_spec = pl.BlockSpec((D, D), lambda i: (0, 0))
_b_spec = pl.BlockSpec((1, D), lambda i: (0, 0))
_grid = (N_PAD // BLK,)

_mm1 = pl.pallas_call(
    _mm1_body,
    grid=_grid,
    in_specs=[_cnt_spec, _row_spec, _w_spec],
    out_specs=_row_spec,
    out_shape=jax.ShapeDtypeStruct((N_PAD, D), jnp.float32),
)

_mid = pl.pallas_call(
    _mid_body,
    grid=_grid,
    in_specs=[_cnt_spec, _par_spec, _b_spec, _w_spec],
    out_specs=[_row_spec, _row_spec],
    out_shape=[
        jax.ShapeDtypeStruct((N_PAD, D), jnp.float32),
        jax.ShapeDtypeStruct((N_PAD, D), jnp.float32),
    ],
)

_fin = pl.pallas_call(
    _fin_body,
    grid=_grid,
    in_specs=[_cnt_spec, _par_spec, _b_spec],
    out_specs=_row_spec,
    out_shape=jax.ShapeDtypeStruct((N_PAD, D), jnp.float32),
)


@jax.jit
def kernel(x, edge_index, W1, b1, W2, b2):
    n_fill = E_PAD - N_EDGES
    fill = jnp.arange(n_fill, dtype=jnp.int32)
    src = jnp.concatenate(
        [edge_index[0], fill % N_NODES]
    ).reshape(NW, CH, CHUNK)
    dst = jnp.concatenate(
        [edge_index[1], N_NODES + (fill % (N_PAD - N_NODES))]
    ).reshape(NW, CH, CHUNK)
    x_pad = jnp.zeros((N_PAD, D), jnp.float32).at[:N_NODES].set(x)
    zeros_init = jnp.zeros((N_PAD, D), jnp.float32)

    cnt_parts = _count_kernel(dst).reshape(NW, N_PAD)
    h1 = _mm1(cnt_parts, x_pad, W1)
    p1 = _agg_kernel(h1, src, dst, zeros_init)
    x1_pad, h2 = _mid(cnt_parts, p1, b1.reshape(1, D), W2)
    p2 = _agg_kernel(h2, src, dst, zeros_init)
    x2_pad = _fin(cnt_parts, p2, b2.reshape(1, D))
    return (x1_pad[:N_NODES], x2_pad[:N_NODES])


# speedup vs baseline: 3.4651x; 1.0109x over previous
import functools
import jax
import jax.numpy as jnp
from jax import lax
from jax.experimental import pallas as pl
from jax.experimental.pallas import tpu as pltpu
from jax.experimental.pallas import tpu_sc as plsc

N_NODES = 10000
N_EDGES = 320000
D = 128

NC = 2
NS = 16
NW = NC * NS
L = 16

N_PAD = 10240
CHUNK = 128
CH = 80
E_PAD = NW * CH * CHUNK
RPT = N_PAD // NS
W = 40
NWIN = CH // W
NBUF = 2

_mesh = plsc.VectorSubcoreMesh(core_axis_name="c", subcore_axis_name="s")


@functools.partial(
    pl.kernel,
    out_type=jax.ShapeDtypeStruct((NW * N_PAD,), jnp.float32),
    mesh=_mesh,
    scratch_types=[
        pltpu.VMEM((CH, CHUNK), jnp.int32),
        pltpu.VMEM((N_PAD,), jnp.float32),
    ],
    compiler_params=pltpu.CompilerParams(needs_layout_passes=False),
)
def _count_kernel(dst_hbm, out_hbm, dst_v, cnt_v):
    w = lax.axis_index("s") * NC + lax.axis_index("c")
    pltpu.sync_copy(dst_hbm.at[w], dst_v)

    zero16 = jnp.zeros((L,), jnp.float32)

    def zbody(i, _):
        cnt_v[pl.ds(i * L, L)] = zero16
        return 0

    lax.fori_loop(0, N_PAD // L, zbody, 0)

    one16 = jnp.ones((L,), jnp.float32)

    def row(j, _):
        def sub(k, _):
            d = dst_v[j, pl.ds(k * L, L)]
            plsc.addupdate_scatter(cnt_v, [d], one16)
            return 0
        lax.fori_loop(0, CHUNK // L, sub, 0)
        return 0

    lax.fori_loop(0, CH, row, 0)
    pltpu.sync_copy(cnt_v, out_hbm.at[pl.ds(w * N_PAD, N_PAD)])


@functools.partial(
    pl.kernel,
    out_type=jax.ShapeDtypeStruct((NC, N_PAD, D), jnp.float32),
    mesh=_mesh,
    scratch_types=[
        pltpu.VMEM((W, CHUNK), jnp.int32),
        pltpu.VMEM((W, CHUNK), jnp.int32),
        pltpu.VMEM((NBUF, CHUNK, D), jnp.float32),
        pltpu.VMEM_SHARED((N_PAD, D), jnp.float32),
        pltpu.SemaphoreType.DMA,
        pltpu.SemaphoreType.DMA,
    ],
)
def _agg_kernel(h_hbm, src_hbm, dst_hbm, zero_hbm, out_hbm,
                src_w, dst_w, rows_v, acc, sg0, sg1):
    c = lax.axis_index("c")
    s = lax.axis_index("s")
    w = s * NC + c
    sg = (sg0, sg1)

    def load_window(win):
        pltpu.sync_copy(src_hbm.at[w, pl.ds(win * W, W)], src_w)
        pltpu.sync_copy(dst_hbm.at[w, pl.ds(win * W, W)], dst_w)
        for b in range(NBUF):
            pltpu.async_copy(h_hbm.at[src_w.at[b]], rows_v.at[b], sg[b])

    load_window(0)
    @pl.when(c == 0)
    def _():
        pltpu.sync_copy(h_hbm.at[pl.ds(s * RPT, RPT)], acc.at[pl.ds(s * RPT, RPT)])

    @pl.when(c != 0)
    def _():
        pltpu.sync_copy(zero_hbm.at[pl.ds(s * RPT, RPT)], acc.at[pl.ds(s * RPT, RPT)])

    plsc.subcore_barrier()

    for win in range(NWIN):
        if win > 0:
            load_window(win)

        def inner(t, _):
            j0 = t * NBUF
            for b in range(NBUF):
                j = j0 + b
                pltpu.make_async_copy(
                    h_hbm.at[src_w.at[j]], rows_v.at[b], sg[b]
                ).wait()
                pltpu.sync_copy(rows_v.at[b], acc.at[dst_w.at[j]], add=True)

                @pl.when(j + NBUF < W)
                def _():
                    pltpu.async_copy(
                        h_hbm.at[src_w.at[j + NBUF]], rows_v.at[b], sg[b]
                    )
            return 0

        lax.fori_loop(0, W // NBUF, inner, 0)
    plsc.subcore_barrier()
    pltpu.sync_copy(acc.at[pl.ds(s * RPT, RPT)], out_hbm.at[c, pl.ds(s * RPT, RPT)])


BLK = 1024


def _dinv_of(cnt_blk):
    deg = jnp.sum(cnt_blk, axis=0) + 1.0
    return lax.rsqrt(deg)


def _mm1_body(cnt_ref, x_ref, w_ref, h_ref):
    dinv = _dinv_of(cnt_ref[...])
    h = jnp.dot(x_ref[...], w_ref[...], preferred_element_type=jnp.float32)
    h_ref[...] = h * dinv[:, None]


def _mid_body(cnt_ref, p_ref, b1_ref, w_ref, x1_ref, h2_ref):
    dinv = _dinv_of(cnt_ref[...])
    agg = p_ref[0] + p_ref[1]
    x1 = jnp.maximum(agg * dinv[:, None] + b1_ref[...], 0.0)
    x1_ref[...] = x1
    h2 = jnp.dot(x1, w_ref[...], preferred_element_type=jnp.float32)
    h2_ref[...] = h2 * dinv[:, None]


def _fin_body(cnt_ref, p_ref, b2_ref, x2_ref):
    dinv = _dinv_of(cnt_ref[...])
    agg = p_ref[0] + p_ref[1]
    x2_ref[...] = agg * dinv[:, None] + b2_ref[...]


_cnt_spec = pl.BlockSpec((NW, BLK), lambda i: (0, i))
_row_spec = pl.BlockSpec((BLK, D), lambda i: (i, 0))
_par_spec = pl.BlockSpec((NC, BLK, D), lambda i: (0, i, 0))
_w_spec = pl.BlockSpec((D, D), lambda i: (0, 0))
_b_spec = pl.BlockSpec((1, D), lambda i: (0, 0))
_grid = (N_PAD // BLK,)

_mm1 = pl.pallas_call(
    _mm1_body,
    grid=_grid,
    in_specs=[_cnt_spec, _row_spec, _w_spec],
    out_specs=_row_spec,
    out_shape=jax.ShapeDtypeStruct((N_PAD, D), jnp.float32),
)

_mid = pl.pallas_call(
    _mid_body,
    grid=_grid,
    in_specs=[_cnt_spec, _par_spec, _b_spec, _w_spec],
    out_specs=[_row_spec, _row_spec],
    out_shape=[
        jax.ShapeDtypeStruct((N_PAD, D), jnp.float32),
        jax.ShapeDtypeStruct((N_PAD, D), jnp.float32),
    ],
)

_fin = pl.pallas_call(
    _fin_body,
    grid=_grid,
    in_specs=[_cnt_spec, _par_spec, _b_spec],
    out_specs=_row_spec,
    out_shape=jax.ShapeDtypeStruct((N_PAD, D), jnp.float32),
)


@jax.jit
def kernel(x, edge_index, W1, b1, W2, b2):
    n_fill = E_PAD - N_EDGES
    fill = jnp.arange(n_fill, dtype=jnp.int32)
    src = jnp.concatenate(
        [edge_index[0], fill % N_NODES]
    ).reshape(NW, CH, CHUNK)
    dst = jnp.concatenate(
        [edge_index[1], N_NODES + (fill % (N_PAD - N_NODES))]
    ).reshape(NW, CH, CHUNK)
    x_pad = jnp.zeros((N_PAD, D), jnp.float32).at[:N_NODES].set(x)
    zeros_init = jnp.zeros((N_PAD, D), jnp.float32)

    cnt_parts = _count_kernel(dst).reshape(NW, N_PAD)
    h1 = _mm1(cnt_parts, x_pad, W1)
    p1 = _agg_kernel(h1, src, dst, zeros_init)
    x1_pad, h2 = _mid(cnt_parts, p1, b1.reshape(1, D), W2)
    p2 = _agg_kernel(h2, src, dst, zeros_init)
    x2_pad = _fin(cnt_parts, p2, b2.reshape(1, D))
    return (x1_pad[:N_NODES], x2_pad[:N_NODES])
